# baseline (speedup 1.0000x reference)
import functools

import jax
import jax.numpy as jnp
from jax import lax
from jax.experimental import pallas as pl
from jax.experimental.pallas import tpu as pltpu
from jax.experimental.pallas import tpu_sc as plsc

_NC = 2
_NS = 16
_NW = _NC * _NS
_NL = 16

_SHIFT = 16384


def _sc_embed_sums(weight, idx2):
    QH, RW = idx2.shape
    V, D = weight.shape
    RT = QH // _NW
    QT = 2 * RT
    ND = D // _NL
    L2 = RW // 2

    mesh = plsc.VectorSubcoreMesh(
        core_axis_name="c", subcore_axis_name="s", num_cores=_NC)

    @functools.partial(
        pl.kernel,
        mesh=mesh,
        out_type=jax.ShapeDtypeStruct((2 * QH, D), jnp.float32),
        scratch_types=[
            pltpu.VMEM((RT, RW), jnp.int32),
            pltpu.VMEM((RW, D), jnp.float32),
            pltpu.VMEM((RW, D), jnp.float32),
            pltpu.VMEM((RW, D), jnp.float32),
            pltpu.VMEM((RW, D), jnp.float32),
            pltpu.VMEM((QT, D), jnp.float32),
            pltpu.SemaphoreType.DMA,
            pltpu.SemaphoreType.DMA,
            pltpu.SemaphoreType.DMA,
            pltpu.SemaphoreType.DMA,
        ],
    )
    def sc_kernel(w_hbm, idx_hbm, out_hbm, idx_all, rows0, rows1, rows2,
                  rows3, out_v, sem0, sem1, sem2, sem3):
        wid = lax.axis_index("s") * _NC + lax.axis_index("c")
        base = wid * RT
        pltpu.sync_copy(idx_hbm.at[pl.ds(base, RT)], idx_all)
        bufs = (rows0, rows1, rows2, rows3)
        sems = (sem0, sem1, sem2, sem3)
        for b in range(3):
            pltpu.async_copy(w_hbm.at[idx_all.at[b]], bufs[b], sems[b])

        def quad(t, carry):
            for b in range(4):
                j = 4 * t + b
                jn = jnp.minimum(j + 3, RT - 1)
                pltpu.async_copy(w_hbm.at[idx_all.at[jn]], bufs[(b + 3) % 4],
                                 sems[(b + 3) % 4])
                pltpu.make_async_copy(w_hbm.at[pl.ds(0, RW)], bufs[b],
                                      sems[b]).wait()
                rv = bufs[b]
                for q in range(2):
                    sls = [pl.ds(d * _NL, _NL) for d in range(ND)]
                    accs = [rv[q * L2, sl] for sl in sls]
                    for l in range(1, L2):
                        for d in range(ND):
                            accs[d] = accs[d] + rv[q * L2 + l, sls[d]]
                    for d in range(ND):
                        out_v[2 * j + q, sls[d]] = accs[d]
            return carry

        lax.fori_loop(0, RT // 4, quad, 0)
        for b in range(3):
            pltpu.make_async_copy(w_hbm.at[pl.ds(0, RW)], bufs[b],
                                  sems[b]).wait()
        pltpu.sync_copy(out_v, out_hbm.at[pl.ds(wid * QT, QT)])

    return sc_kernel(weight, idx2)


def _tc_cosine_argmax(q_sums, w0, train, y3, seq_len, npad, kb):
    Q, D = q_sums.shape
    K, _ = train.shape
    nkb = K // kb
    imax = 2**31 - 1

    def body(qs_ref, w0_ref, tr_ref, y_ref, sc_ref, yp_ref, qn_ref, bp_ref,
             sims_a, sims_b):
        kk = pl.program_id(0)

        @pl.when(kk == 0)
        def _init():
            q = (qs_ref[...] - jnp.float32(npad) * w0_ref[...]) * jnp.float32(1.0 / seq_len)
            n = jnp.sqrt(jnp.sum(q * q, axis=1, keepdims=True))
            qn_ref[...] = q / jnp.maximum(n, 1e-8)
            sc_ref[...] = jnp.full((Q, 1), -3.0, jnp.float32)
            bp_ref[...] = jnp.full((Q, 1), 2.0**24, jnp.float32)

        def stage(wbuf, rbuf):
            kblk = tr_ref[...]
            kn2 = jnp.sqrt(jnp.sum(kblk * kblk, axis=1, keepdims=True))
            kn = kblk / jnp.maximum(kn2, 1e-8)
            wbuf[...] = lax.dot_general(
                qn_ref[...], kn, (((1,), (1,)), ((), ())),
                preferred_element_type=jnp.float32)
            sims = rbuf[...]
            bm = jnp.max(sims, axis=1, keepdims=True)
            pcol = (lax.broadcasted_iota(jnp.int32, (1, kb), 1).astype(jnp.float32)
                    * _SHIFT + y_ref[0].astype(jnp.float32))
            cand = jnp.where(sims == bm, jnp.broadcast_to(pcol, sims.shape),
                             jnp.float32(2.0**24))
            bp = jnp.min(cand, axis=1, keepdims=True)
            better = jnp.logical_and(bm > sc_ref[...], kk > 0)
            sc_ref[...] = jnp.where(better, bm, sc_ref[...])
            new_p = jnp.where(better, bp, bp_ref[...])
            bp_ref[...] = new_p
            yp_ref[...] = jnp.bitwise_and(new_p.astype(jnp.int32), _SHIFT - 1)

        @pl.when(kk % 2 == 0)
        def _even():
            stage(sims_a, sims_b)

        @pl.when(kk % 2 == 1)
        def _odd():
            stage(sims_b, sims_a)

    last = nkb - 1
    return pl.pallas_call(
        body,
        grid=(nkb + 1,),
        in_specs=[
            pl.BlockSpec((Q, D), lambda k: (0, 0)),
            pl.BlockSpec((1, D), lambda k: (0, 0)),
            pl.BlockSpec((kb, D), lambda k: (jnp.minimum(k, last), 0)),
            pl.BlockSpec((1, 1, kb), lambda k: (jnp.maximum(k - 1, 0), 0, 0)),
        ],
        out_specs=[
            pl.BlockSpec((Q, 1), lambda k: (0, 0)),
            pl.BlockSpec((Q, 1), lambda k: (0, 0)),
        ],
        out_shape=[
            jax.ShapeDtypeStruct((Q, 1), jnp.float32),
            jax.ShapeDtypeStruct((Q, 1), jnp.int32),
        ],
        scratch_shapes=[pltpu.VMEM((Q, D), jnp.float32),
                        pltpu.VMEM((Q, 1), jnp.float32),
                        pltpu.VMEM((Q, kb), jnp.float32),
                        pltpu.VMEM((Q, kb), jnp.float32)],
        compiler_params=pltpu.CompilerParams(
            dimension_semantics=("arbitrary",)),
    )(q_sums, w0, train, y3)


def kernel(insts, weight, train_embs, y_train):
    insts = insts.astype(jnp.int32)
    y_train = y_train.astype(jnp.int32)
    L, Q = insts.shape
    V, D = weight.shape
    K, _ = train_embs.shape

    LP = -(-L // 4) * 4
    idx = jnp.transpose(insts)
    idx = jnp.pad(idx, ((0, 0), (0, LP - L)))
    idx2 = idx.reshape(Q // 2, 2 * LP)

    kb = max(d for d in range(min(K, 1024), 0, -1) if K % d == 0)
    y3 = y_train.reshape(K // kb, 1, kb)

    q_sums = _sc_embed_sums(weight, idx2)

    nch = 4 if Q % 4 == 0 else 1
    qc = Q // nch
    parts = [_tc_cosine_argmax(q_sums[i * qc:(i + 1) * qc], weight[0:1],
                               train_embs, y3, L, LP - L, kb)
             for i in range(nch)]
    scores = jnp.concatenate([p[0].reshape(-1) for p in parts])
    ypred = jnp.concatenate([p[1].reshape(-1) for p in parts])
    return scores, ypred


# speedup vs baseline: 1.3272x; 1.3272x over previous
import functools

import jax
import jax.numpy as jnp
from jax import lax
from jax.experimental import pallas as pl
from jax.experimental.pallas import tpu as pltpu
from jax.experimental.pallas import tpu_sc as plsc

_NC = 2
_NS = 16
_NW = _NC * _NS
_NL = 16

_SHIFT = 1024


def _sc_embed_sums(weight, idx2):
    QH, RW = idx2.shape
    V, D = weight.shape
    RT = QH // _NW
    QT = 2 * RT
    ND = D // _NL
    L2 = RW // 2

    mesh = plsc.VectorSubcoreMesh(
        core_axis_name="c", subcore_axis_name="s", num_cores=_NC)

    @functools.partial(
        pl.kernel,
        mesh=mesh,
        out_type=jax.ShapeDtypeStruct((2 * QH, D), jnp.float32),
        scratch_types=[
            pltpu.VMEM((RT, RW), jnp.int32),
            pltpu.VMEM((RW, D), jnp.float32),
            pltpu.VMEM((RW, D), jnp.float32),
            pltpu.VMEM((RW, D), jnp.float32),
            pltpu.VMEM((RW, D), jnp.float32),
            pltpu.VMEM((QT, D), jnp.float32),
            pltpu.SemaphoreType.DMA,
            pltpu.SemaphoreType.DMA,
            pltpu.SemaphoreType.DMA,
            pltpu.SemaphoreType.DMA,
        ],
    )
    def sc_kernel(w_hbm, idx_hbm, out_hbm, idx_all, rows0, rows1, rows2,
                  rows3, out_v, sem0, sem1, sem2, sem3):
        wid = lax.axis_index("s") * _NC + lax.axis_index("c")
        base = wid * RT
        pltpu.sync_copy(idx_hbm.at[pl.ds(base, RT)], idx_all)
        bufs = (rows0, rows1, rows2, rows3)
        sems = (sem0, sem1, sem2, sem3)
        for b in range(3):
            pltpu.async_copy(w_hbm.at[idx_all.at[b]], bufs[b], sems[b])

        def quad(t, carry):
            for b in range(4):
                j = 4 * t + b
                jn = jnp.minimum(j + 3, RT - 1)
                pltpu.async_copy(w_hbm.at[idx_all.at[jn]], bufs[(b + 3) % 4],
                                 sems[(b + 3) % 4])
                pltpu.make_async_copy(w_hbm.at[pl.ds(0, RW)], bufs[b],
                                      sems[b]).wait()
                rv = bufs[b]
                for q in range(2):
                    sls = [pl.ds(d * _NL, _NL) for d in range(ND)]
                    accs = [rv[q * L2, sl] for sl in sls]
                    for l in range(1, L2):
                        for d in range(ND):
                            accs[d] = accs[d] + rv[q * L2 + l, sls[d]]
                    for d in range(ND):
                        out_v[2 * j + q, sls[d]] = accs[d]
            return carry

        lax.fori_loop(0, RT // 4, quad, 0)
        for b in range(3):
            pltpu.make_async_copy(w_hbm.at[pl.ds(0, RW)], bufs[b],
                                  sems[b]).wait()
        pltpu.sync_copy(out_v, out_hbm.at[pl.ds(wid * QT, QT)])

    return sc_kernel(weight, idx2)


def _tc_cosine_argmax(q_sums, w0, train, y3, seq_len, npad, kb):
    Q, D = q_sums.shape
    K, _ = train.shape
    nkb = K // kb
    imax = 2**31 - 1

    def body(qs_ref, w0_ref, tr_ref, y_ref, sc_ref, yp_ref, qn_ref, bp_ref,
             sims_a, sims_b):
        kk = pl.program_id(0)

        @pl.when(kk == 0)
        def _init():
            q = (qs_ref[...] - jnp.float32(npad) * w0_ref[...]) * jnp.float32(1.0 / seq_len)
            n = jnp.sqrt(jnp.sum(q * q, axis=1, keepdims=True))
            qn_ref[...] = q / jnp.maximum(n, 1e-8)
            sc_ref[...] = jnp.full((Q, 1), -3.0, jnp.float32)
            bp_ref[...] = jnp.full((Q, 1), 2.0**24, jnp.float32)

        def stage(wbuf, rbuf):
            kblk = tr_ref[...]
            kn2 = jnp.sqrt(jnp.sum(kblk * kblk, axis=1, keepdims=True))
            kn = kblk / jnp.maximum(kn2, 1e-8)
            wbuf[...] = lax.dot_general(
                qn_ref[...], kn, (((1,), (1,)), ((), ())),
                preferred_element_type=jnp.float32)
            sims = rbuf[...]
            bm = jnp.max(sims, axis=1, keepdims=True)
            pcol = (lax.broadcasted_iota(jnp.int32, (1, kb), 1).astype(jnp.float32)
                    * _SHIFT + y_ref[0].astype(jnp.float32))
            cand = jnp.where(sims == bm, pcol, jnp.float32(2.0**24))
            bp = jnp.min(cand, axis=1, keepdims=True)
            better = jnp.logical_and(bm > sc_ref[...], kk > 0)
            sc_ref[...] = jnp.where(better, bm, sc_ref[...])
            new_p = jnp.where(better, bp, bp_ref[...])
            bp_ref[...] = new_p
            yp_ref[...] = jnp.bitwise_and(new_p.astype(jnp.int32), _SHIFT - 1)

        @pl.when(kk % 2 == 0)
        def _even():
            stage(sims_a, sims_b)

        @pl.when(kk % 2 == 1)
        def _odd():
            stage(sims_b, sims_a)

    last = nkb - 1
    return pl.pallas_call(
        body,
        grid=(nkb + 1,),
        in_specs=[
            pl.BlockSpec((Q, D), lambda k: (0, 0)),
            pl.BlockSpec((1, D), lambda k: (0, 0)),
            pl.BlockSpec((kb, D), lambda k: (jnp.minimum(k, last), 0)),
            pl.BlockSpec((1, 1, kb), lambda k: (jnp.maximum(k - 1, 0), 0, 0)),
        ],
        out_specs=[
            pl.BlockSpec((Q, 1), lambda k: (0, 0)),
            pl.BlockSpec((Q, 1), lambda k: (0, 0)),
        ],
        out_shape=[
            jax.ShapeDtypeStruct((Q, 1), jnp.float32),
            jax.ShapeDtypeStruct((Q, 1), jnp.int32),
        ],
        scratch_shapes=[pltpu.VMEM((Q, D), jnp.float32),
                        pltpu.VMEM((Q, 1), jnp.float32),
                        pltpu.VMEM((Q, kb), jnp.float32),
                        pltpu.VMEM((Q, kb), jnp.float32)],
        compiler_params=pltpu.CompilerParams(
            dimension_semantics=("arbitrary",)),
    )(q_sums, w0, train, y3)


def kernel(insts, weight, train_embs, y_train):
    insts = insts.astype(jnp.int32)
    y_train = y_train.astype(jnp.int32)
    L, Q = insts.shape
    V, D = weight.shape
    K, _ = train_embs.shape

    LP = -(-L // 4) * 4
    idx = jnp.transpose(insts)
    idx = jnp.pad(idx, ((0, 0), (0, LP - L)))
    idx2 = idx.reshape(Q // 2, 2 * LP)

    kb = max(d for d in range(min(K, 2048), 0, -1)
             if K % d == 0 and d * _SHIFT + _SHIFT <= 2**24)
    y3 = y_train.reshape(K // kb, 1, kb)

    nch = 4 if (Q // 2) % (4 * _NW) == 0 else 1
    qh = Q // 2 // nch
    sums = [_sc_embed_sums(weight, idx2[i * qh:(i + 1) * qh])
            for i in range(nch)]
    parts = [_tc_cosine_argmax(s, weight[0:1], train_embs, y3, L, LP - L, kb)
             for s in sums]
    scores = jnp.concatenate([p[0].reshape(-1) for p in parts])
    ypred = jnp.concatenate([p[1].reshape(-1) for p in parts])
    return scores, ypred
